# baseline (speedup 1.0000x reference)
import functools
import numpy as np
import jax
import jax.numpy as jnp
from jax import lax
from jax.experimental import pallas as pl
from jax.experimental.pallas import tpu as pltpu
from jax.experimental.pallas import tpu_sc as plsc

_F = 26
_NUM_TOKENS = 8
_VOCAB = 100000
_EMB = 16
_NT = 2
_DM = 128
_B = 4096

_PERM = np.random.default_rng(0).permutation(_F).tolist()
_GROUPS = [
    _PERM[_F * i // _NUM_TOKENS: _F * (i + 1) // _NUM_TOKENS]
    for i in range(_NUM_TOKENS)
]
_GSIZES = [len(g) for g in _GROUPS]
_GCUM = np.cumsum([0] + _GSIZES).tolist()
_J = 2 * _F

_ORDER_FEATS = np.array(
    [f for k in range(_NUM_TOKENS) for t in range(_NT) for f in _GROUPS[k]],
    dtype=np.int32)
_BASES = np.array(
    [(t * _F + f) * _VOCAB
     for k in range(_NUM_TOKENS) for t in range(_NT) for f in _GROUPS[k]],
    dtype=np.int32)

_NC = 2
_NS = 16
_NW = _NC * _NS
_RW = _B // _NW
_CB = 64
_NCHUNK = _RW // _CB


def _sc_gather_pool(idx_flat, table_flat, interpret=False):
    mesh = plsc.VectorSubcoreMesh(core_axis_name="c", subcore_axis_name="s")

    @functools.partial(
        pl.kernel,
        out_type=jax.ShapeDtypeStruct((_B, 16 * _NT * _NUM_TOKENS), jnp.float32),
        mesh=mesh,
        scratch_types=[
            pltpu.VMEM((_CB * _J,), jnp.int32),
            pltpu.VMEM((_CB * _J, _EMB), jnp.float32),
            pltpu.VMEM((_CB, 16 * _NT * _NUM_TOKENS), jnp.float32),
            pltpu.SemaphoreType.DMA,
        ],
        compiler_params=pltpu.CompilerParams(use_tc_tiling_on_sc=False),
        interpret=interpret,
    )
    def sc_kernel(idx_hbm, tab_hbm, out_hbm, idx_v, rows_v, pooled_v, sem):
        wid = lax.axis_index("s") * _NC + lax.axis_index("c")
        for c in range(_NCHUNK):
            base_row = wid * _RW + c * _CB
            pltpu.sync_copy(idx_hbm.at[pl.ds(base_row * _J, _CB * _J)], idx_v)
            pltpu.async_copy(tab_hbm.at[idx_v], rows_v, sem).wait()

            def pool_body(r, carry):
                j0 = r * _J
                for k in range(_NUM_TOKENS):
                    n = _GSIZES[k]
                    for t in range(_NT):
                        s0 = 2 * _GCUM[k] + t * n
                        acc = rows_v[j0 + s0, :]
                        for i in range(1, n):
                            acc = acc + rows_v[j0 + s0 + i, :]
                        pooled_v[r, pl.ds((k * _NT + t) * 16, 16)] = (
                            acc * (1.0 / n))
                return carry

            lax.fori_loop(0, _CB, pool_body, 0)
            pltpu.sync_copy(pooled_v, out_hbm.at[pl.ds(base_row, _CB)])

    return sc_kernel(idx_flat, table_flat)


_TB = 512


def _tc_body(pooled_ref, mask_ref, emat_ref, w_ref, b_ref, g_ref, bb_ref,
             out_ref):
    mc = jnp.dot(mask_ref[:], emat_ref[:], preferred_element_type=jnp.float32)
    x = pooled_ref[:] + mc
    w = w_ref[:]
    b = b_ref[:]
    gam = g_ref[:]
    bet = bb_ref[:]
    for k in range(_NUM_TOKENS):
        xk = x[:, k * 32:(k + 1) * 32]
        yk = jnp.dot(xk, w, preferred_element_type=jnp.float32) + b
        yk = yk * jax.nn.sigmoid(yk)
        mu = jnp.mean(yk, axis=1, keepdims=True)
        d = yk - mu
        var = jnp.mean(d * d, axis=1, keepdims=True)
        out_ref[:, k, :] = d * lax.rsqrt(var + 1e-5) * gam + bet


def _tc_tail(pooled, missing_mask, emat, w_t, b2, g2, bb2, interpret=False):
    grid = (_B // _TB,)
    return pl.pallas_call(
        _tc_body,
        grid=grid,
        in_specs=[
            pl.BlockSpec((_TB, 16 * _NT * _NUM_TOKENS), lambda i: (i, 0)),
            pl.BlockSpec((_TB, _F), lambda i: (i, 0)),
            pl.BlockSpec((_F, 16 * _NT * _NUM_TOKENS), lambda i: (0, 0)),
            pl.BlockSpec((2 * _EMB, _DM), lambda i: (0, 0)),
            pl.BlockSpec((1, _DM), lambda i: (0, 0)),
            pl.BlockSpec((1, _DM), lambda i: (0, 0)),
            pl.BlockSpec((1, _DM), lambda i: (0, 0)),
        ],
        out_specs=pl.BlockSpec((_TB, _NUM_TOKENS, _DM), lambda i: (i, 0, 0)),
        out_shape=jax.ShapeDtypeStruct((_B, _NUM_TOKENS, _DM), jnp.float32),
        interpret=interpret,
    )(pooled, missing_mask, emat, w_t, b2, g2, bb2)


def _build_emat(missing_emb):
    h = np.zeros((_F, _NUM_TOKENS), dtype=np.float32)
    for k, g in enumerate(_GROUPS):
        for f in g:
            h[f, k] = 1.0 / _GSIZES[k]
    me_f = jnp.transpose(missing_emb, (1, 0, 2))
    emat = h[:, :, None, None] * me_f[:, None, :, :]
    return emat.reshape(_F, _NUM_TOKENS * _NT * _EMB)


def kernel(int_feats, missing_mask, tables, missing_emb, W_proj, b_proj,
           ln_gamma, ln_beta):
    idx = int_feats[:, _ORDER_FEATS] + _BASES[None, :]
    idx_flat = idx.reshape(_B * _J)
    table_flat = tables.reshape(_NT * _F * _VOCAB, _EMB)
    pooled = _sc_gather_pool(idx_flat, table_flat)
    emat = _build_emat(missing_emb)
    return _tc_tail(pooled, missing_mask, emat, W_proj.T,
                    b_proj.reshape(1, _DM), ln_gamma.reshape(1, _DM),
                    ln_beta.reshape(1, _DM))
